# Initial kernel scaffold; baseline (speedup 1.0000x reference)
import jax
import jax.numpy as jnp
from jax.experimental import pallas as pl


def kernel(x, edge_index, W1, b1, W2, b2, Wfc, bfc):
    raise NotImplementedError("write your pallas kernel here")


# speedup vs baseline: 16.3799x; 16.3799x over previous
import jax
import jax.numpy as jnp
from jax import lax
from jax.experimental import pallas as pl
from jax.experimental.pallas import tpu as pltpu
from jax.experimental.pallas import tpu_sc as plsc

N = 10000
E = 320000
D = 128
HD = D // 2
OUT = 11

NC = 2
NS = 16
NW = NC * NS
K = 80
NCHD = E // NW // K
NCHS = E // NS // K
NPAD = 10240
DPT = NPAD // NS
ZR = 128
DW = 16

RB = 400
GRID = N // RB


def _deg_body(dst_hbm, deg_out, dst_v, ones_v, zv, deg_sh):
    c = lax.axis_index("c")
    s = lax.axis_index("s")
    w = s * NC + c

    def fill_ones(i, carry):
        ones_v[i, pl.ds(0, DW)] = jnp.ones((DW,), jnp.float32)
        return carry

    lax.fori_loop(0, K, fill_ones, 0)

    def fill_zero(i, carry):
        zv[i, pl.ds(0, DW)] = jnp.zeros((DW,), jnp.float32)
        return carry

    lax.fori_loop(0, DPT, fill_zero, 0)
    pltpu.sync_copy(zv, deg_sh.at[pl.ds(s * DPT, DPT)])
    pltpu.sync_copy(dst_hbm.at[w], dst_v)
    plsc.subcore_barrier()

    def chunk(j, carry):
        pltpu.sync_copy(ones_v, deg_sh.at[dst_v.at[j]], add=True)
        return carry

    lax.fori_loop(0, NCHD, chunk, 0)
    plsc.subcore_barrier()
    pltpu.sync_copy(deg_sh.at[pl.ds(s * DPT, DPT)],
                    deg_out.at[c, pl.ds(s * DPT, DPT)])


def _scatter_body(g_hbm, src_hbm, dst_hbm, acc_out,
                  src_v, dst_v, rows0, rows1, zbuf, acc_sh, sem0, sem1):
    c = lax.axis_index("c")
    s = lax.axis_index("s")

    def zrow(i, carry):
        for l in range(HD // 16):
            zbuf[i, pl.ds(l * 16, 16)] = jnp.zeros((16,), jnp.float32)
        return carry

    lax.fori_loop(0, ZR, zrow, 0)
    for i in range(DPT // ZR):
        pltpu.sync_copy(zbuf, acc_sh.at[pl.ds(s * DPT + i * ZR, ZR)])
    pltpu.sync_copy(src_hbm.at[s], src_v)
    pltpu.sync_copy(dst_hbm.at[s], dst_v)
    plsc.subcore_barrier()

    gsrc = g_hbm.at[c]

    pltpu.async_copy(gsrc.at[src_v.at[0]], rows0, sem0)

    def pair(p, carry):
        j = 2 * p
        pltpu.make_async_copy(gsrc.at[src_v.at[0]], rows0, sem0).wait()
        pltpu.async_copy(gsrc.at[src_v.at[j + 1]], rows1, sem1)
        pltpu.sync_copy(rows0, acc_sh.at[dst_v.at[j]], add=True)
        pltpu.make_async_copy(gsrc.at[src_v.at[0]], rows1, sem1).wait()
        pltpu.async_copy(gsrc.at[src_v.at[j + 2]], rows0, sem0)
        pltpu.sync_copy(rows1, acc_sh.at[dst_v.at[j + 1]], add=True)
        return carry

    lax.fori_loop(0, NCHS // 2 - 1, pair, 0)
    j = NCHS - 2
    pltpu.make_async_copy(gsrc.at[src_v.at[0]], rows0, sem0).wait()
    pltpu.async_copy(gsrc.at[src_v.at[j + 1]], rows1, sem1)
    pltpu.sync_copy(rows0, acc_sh.at[dst_v.at[j]], add=True)
    pltpu.make_async_copy(gsrc.at[src_v.at[0]], rows1, sem1).wait()
    pltpu.sync_copy(rows1, acc_sh.at[dst_v.at[j + 1]], add=True)
    plsc.subcore_barrier()
    for i in range(DPT // ZR):
        pltpu.sync_copy(acc_sh.at[pl.ds(s * DPT + i * ZR, ZR)],
                        acc_out.at[c, pl.ds(s * DPT + i * ZR, ZR)])


def _sc_mesh():
    return plsc.VectorSubcoreMesh(core_axis_name="c", subcore_axis_name="s",
                                  num_cores=NC, num_subcores=NS)


def _deg_call(dst_r):
    f = pl.kernel(
        _deg_body,
        out_type=jax.ShapeDtypeStruct((NC, NPAD, DW), jnp.float32),
        mesh=_sc_mesh(),
        compiler_params=pltpu.CompilerParams(use_tc_tiling_on_sc=False),
        scratch_types=[
            pltpu.VMEM((NCHD, K), jnp.int32),
            pltpu.VMEM((K, DW), jnp.float32),
            pltpu.VMEM((DPT, DW), jnp.float32),
            pltpu.VMEM_SHARED((NPAD, DW), jnp.float32),
        ],
    )
    return f(dst_r)


def _scatter_call(g, src_r, dst_r):
    f = pl.kernel(
        _scatter_body,
        out_type=jax.ShapeDtypeStruct((NC, NPAD, HD), jnp.float32),
        mesh=_sc_mesh(),
        compiler_params=pltpu.CompilerParams(use_tc_tiling_on_sc=False),
        scratch_types=[
            pltpu.VMEM((NCHS, K), jnp.int32),
            pltpu.VMEM((NCHS, K), jnp.int32),
            pltpu.VMEM((K, HD), jnp.float32),
            pltpu.VMEM((K, HD), jnp.float32),
            pltpu.VMEM((ZR, HD), jnp.float32),
            pltpu.VMEM_SHARED((NPAD, HD), jnp.float32),
            pltpu.SemaphoreType.DMA,
            pltpu.SemaphoreType.DMA,
        ],
    )
    return f(g, src_r, dst_r)


def _l1_body(d0, d1, x, w, o):
    dinv = lax.rsqrt(d0[...] + d1[...] + 1.0)
    h = jnp.dot(x[...], w[...], preferred_element_type=jnp.float32) * dinv
    o[0] = h[:, :HD]
    o[1] = h[:, HD:]


def _l2_body(d0, d1, acc, g, b, w, o):
    dinv = lax.rsqrt(d0[...] + d1[...] + 1.0)
    agg = jnp.concatenate([acc[0] + g[0], acc[1] + g[1]], axis=-1)
    h = jnp.maximum(agg * dinv + b[...], 0.0)
    t = jnp.dot(h, w[...], preferred_element_type=jnp.float32) * dinv
    o[0] = t[:, :HD]
    o[1] = t[:, HD:]


def _out_body(d0, d1, acc, g, b, wfc, bfc, o):
    dinv = lax.rsqrt(d0[...] + d1[...] + 1.0)
    agg = jnp.concatenate([acc[0] + g[0], acc[1] + g[1]], axis=-1)
    h = jnp.maximum(agg * dinv + b[...], 0.0)
    o[...] = jnp.dot(h, wfc[...], preferred_element_type=jnp.float32) + bfc[...]


_D_SPEC = pl.BlockSpec((RB, 1), lambda i: (i, 0))
_ROW_SPEC = pl.BlockSpec((RB, D), lambda i: (i, 0))
_W_SPEC = pl.BlockSpec((D, D), lambda i: (0, 0))
_B_SPEC = pl.BlockSpec((1, D), lambda i: (0, 0))
_SPLIT_SPEC = pl.BlockSpec((NC, RB, HD), lambda i: (0, i, 0))
_O_SPEC = pl.BlockSpec((RB, D), lambda i: (i, 0))


def _l1_call(d0, d1, x, w):
    return pl.pallas_call(
        _l1_body,
        grid=(GRID,),
        in_specs=[_D_SPEC, _D_SPEC, _ROW_SPEC, _W_SPEC],
        out_specs=_SPLIT_SPEC,
        out_shape=jax.ShapeDtypeStruct((NC, N, HD), jnp.float32),
    )(d0, d1, x, w)


def _l2_call(d0, d1, acc, g, b, w):
    return pl.pallas_call(
        _l2_body,
        grid=(GRID,),
        in_specs=[_D_SPEC, _D_SPEC, _SPLIT_SPEC, _SPLIT_SPEC, _B_SPEC, _W_SPEC],
        out_specs=_SPLIT_SPEC,
        out_shape=jax.ShapeDtypeStruct((NC, N, HD), jnp.float32),
    )(d0, d1, acc, g, b, w)


def _out_call(d0, d1, acc, g, b, wfc, bfc):
    return pl.pallas_call(
        _out_body,
        grid=(GRID,),
        in_specs=[_D_SPEC, _D_SPEC, _SPLIT_SPEC, _SPLIT_SPEC, _B_SPEC, _W_SPEC,
                  _B_SPEC],
        out_specs=_O_SPEC,
        out_shape=jax.ShapeDtypeStruct((N, D), jnp.float32),
    )(d0, d1, acc, g, b, wfc, bfc)


def kernel(x, edge_index, W1, b1, W2, b2, Wfc, bfc):
    src_r = edge_index[0].reshape(NS, NCHS, K)
    dst_r = edge_index[1].reshape(NS, NCHS, K)
    dstdeg_r = edge_index[1].reshape(NW, NCHD, K)

    deg = _deg_call(dstdeg_r)
    d0 = deg[0, :N, 0:1]
    d1 = deg[1, :N, 0:1]

    g1 = _l1_call(d0, d1, x, W1)
    acc1 = _scatter_call(g1, src_r, dst_r)
    g2 = _l2_call(d0, d1, acc1, g1, b1.reshape(1, D), W2)
    acc2 = _scatter_call(g2, src_r, dst_r)

    wfc_p = jnp.pad(Wfc, ((0, 0), (0, D - OUT)))
    bfc_p = jnp.pad(bfc, (0, D - OUT)).reshape(1, D)
    out = _out_call(d0, d1, acc2, g2, b2.reshape(1, D), wfc_p, bfc_p)
    return out[:, :OUT]
